# baseline (speedup 1.0000x reference)
import functools

import jax
import jax.numpy as jnp
from jax import lax
from jax.experimental import pallas as pl
from jax.experimental.pallas import tpu as pltpu
from jax.experimental.pallas import tpu_sc as plsc

EMB = 32
NTOK = 16384
LANES = 16

NC = 2
NS = 16
NW = NC * NS
TPW = NTOK // NW
NBUF = 16
NGRP = TPW // NBUF


def _emb_body(tok_hbm, wt_hbm, out_hbm, tok_v, win_s, col_v, out_v, sa, sb):
    sid = lax.axis_index("s")
    wid = sid * NC + lax.axis_index("c")
    base = wid * TPW
    win = win_s.at[sid]
    pltpu.sync_copy(tok_hbm.at[pl.ds(base, TPW)], tok_v)

    e_lo = lax.iota(jnp.int32, LANES)
    e_hi = e_lo + LANES

    def enq_window(t, b):
        c0 = (t // 128) * 128
        pltpu.async_copy(wt_hbm.at[:, pl.ds(c0, 128)], win.at[b], sa.at[b])

    def wait_window(b):
        pltpu.make_async_copy(
            wt_hbm.at[:, pl.ds(0, 128)], win.at[b], sa.at[b]
        ).wait()

    def enq_column(t, b, h, j):
        q16 = ((t - (t // 128) * 128) // 16) * 16
        pltpu.async_copy(
            win.at[b, :, pl.ds(q16, 16)],
            col_v.at[h, :, pl.ds(j * 16, 16)],
            sb,
        )

    def wait_columns(h):
        pltpu.make_async_copy(
            wt_hbm.at[:, pl.ds(0, 128)], col_v.at[h], sb
        ).wait()

    def extract(t, i, h, j):
        lane = j * 16 + (t - (t // 16) * 16)
        lane_v = jnp.full((LANES,), lane, jnp.int32)
        pos_v = jnp.full((LANES,), i, jnp.int32)
        lo = plsc.load_gather(col_v.at[h], [e_lo, lane_v])
        hi = plsc.load_gather(col_v.at[h], [e_hi, lane_v])
        plsc.store_scatter(out_v, [e_lo, pos_v], lo)
        plsc.store_scatter(out_v, [e_hi, pos_v], hi)

    tv0 = tok_v[pl.ds(0, NBUF)]
    for b in range(NBUF):
        enq_window(tv0[b], b)

    def step(g, _):
        tv = tok_v[pl.ds(g * NBUF, NBUF)]
        tnext = tok_v[pl.ds(jnp.minimum(g + 1, NGRP - 1) * NBUF, NBUF)]
        for half in range(2):
            lo_b = half * 8
            for j in range(8):
                b = lo_b + j
                wait_window(b)
                enq_column(tv[b], b, half, j)
            wait_columns(half)
            for j in range(8):
                b = lo_b + j
                extract(tv[b], g * NBUF + b, half, j)

                @pl.when(g + 1 < NGRP)
                def _():
                    enq_window(tnext[b], b)

        return _

    lax.fori_loop(0, NGRP, step, None, unroll=False)

    pltpu.sync_copy(out_v, out_hbm.at[:, pl.ds(base, TPW)])


_emb = functools.partial(
    pl.kernel,
    out_type=jax.ShapeDtypeStruct((EMB, NTOK), jnp.float32),
    mesh=plsc.VectorSubcoreMesh(core_axis_name="c", subcore_axis_name="s"),
    scratch_types=[
        pltpu.VMEM((TPW,), jnp.int32),
        pltpu.VMEM_SHARED((NS, NBUF, EMB, 128), jnp.float32),
        pltpu.VMEM((2, EMB, 128), jnp.float32),
        pltpu.VMEM((EMB, TPW), jnp.float32),
        pltpu.SemaphoreType.DMA((NBUF,)),
        pltpu.SemaphoreType.DMA,
    ],
    compiler_params=pltpu.CompilerParams(needs_layout_passes=False),
)(_emb_body)


def kernel(tokens, weight, bias):
    out_t = _emb(tokens, weight.T)
    out = out_t.T
    return (out, out)


# speedup vs baseline: 1.4944x; 1.4944x over previous
import functools

import jax
import jax.numpy as jnp
from jax import lax
from jax.experimental import pallas as pl
from jax.experimental.pallas import tpu as pltpu
from jax.experimental.pallas import tpu_sc as plsc

EMB = 32
NTOK = 16384
LANES = 16

NC = 2
NS = 16
NW = NC * NS
TPW = NTOK // NW
GRP = 16
HALF = 8
NGRP = TPW // GRP


def _emb_body(tok_hbm, wt_hbm, out_hbm, tok_v, dwin, swin_s, col_v, out_v,
              sa, sw, sb):
    sid = lax.axis_index("s")
    wid = sid * NC + lax.axis_index("c")
    base = wid * TPW
    swin = swin_s.at[sid]
    pltpu.sync_copy(tok_hbm.at[pl.ds(base, TPW)], tok_v)

    e_lo = lax.iota(jnp.int32, LANES)
    e_hi = e_lo + LANES

    def enq_direct(t, b):
        c0 = (t // 128) * 128
        pltpu.async_copy(wt_hbm.at[:, pl.ds(c0, 128)], dwin.at[b], sa.at[b])

    def wait_direct(b):
        pltpu.make_async_copy(
            wt_hbm.at[:, pl.ds(0, 128)], dwin.at[b], sa.at[b]
        ).wait()

    def extract_direct(t, i, b):
        col = t - (t // 128) * 128
        col_vec = jnp.full((LANES,), col, jnp.int32)
        pos_v = jnp.full((LANES,), i, jnp.int32)
        lo = plsc.load_gather(dwin.at[b], [e_lo, col_vec])
        hi = plsc.load_gather(dwin.at[b], [e_hi, col_vec])
        plsc.store_scatter(out_v, [e_lo, pos_v], lo)
        plsc.store_scatter(out_v, [e_hi, pos_v], hi)

    def enq_swin(t, j):
        c0 = (t // 128) * 128
        pltpu.async_copy(wt_hbm.at[:, pl.ds(c0, 128)], swin.at[j], sw.at[j])

    def wait_swin(j):
        pltpu.make_async_copy(
            wt_hbm.at[:, pl.ds(0, 128)], swin.at[j], sw.at[j]
        ).wait()

    def enq_column(t, j):
        q16 = ((t - (t // 128) * 128) // 16) * 16
        pltpu.async_copy(
            swin.at[j, :, pl.ds(q16, 16)],
            col_v.at[:, pl.ds(j * 16, 16)],
            sb,
        )

    def wait_columns():
        pltpu.make_async_copy(
            wt_hbm.at[:, pl.ds(0, 128)], col_v, sb
        ).wait()

    def extract_col(t, i, j):
        lane = j * 16 + (t - (t // 16) * 16)
        lane_v = jnp.full((LANES,), lane, jnp.int32)
        pos_v = jnp.full((LANES,), i, jnp.int32)
        lo = plsc.load_gather(col_v, [e_lo, lane_v])
        hi = plsc.load_gather(col_v, [e_hi, lane_v])
        plsc.store_scatter(out_v, [e_lo, pos_v], lo)
        plsc.store_scatter(out_v, [e_hi, pos_v], hi)

    tv0 = tok_v[pl.ds(0, GRP)]
    for b in range(HALF):
        enq_direct(tv0[b], b)
    for j in range(HALF):
        enq_swin(tv0[HALF + j], j)

    def step(g, _):
        tv = tok_v[pl.ds(g * GRP, GRP)]
        tnext = tok_v[pl.ds(jnp.minimum(g + 1, NGRP - 1) * GRP, GRP)]
        for j in range(HALF):
            wait_swin(j)
            enq_column(tv[HALF + j], j)
        for b in range(HALF):
            i = g * GRP + b
            wait_direct(b)
            extract_direct(tv[b], i, b)

            @pl.when(g + 1 < NGRP)
            def _():
                enq_direct(tnext[b], b)

        wait_columns()
        for j in range(HALF):
            i = g * GRP + HALF + j
            extract_col(tv[HALF + j], i, j)

            @pl.when(g + 1 < NGRP)
            def _():
                enq_swin(tnext[HALF + j], j)

        return _

    lax.fori_loop(0, NGRP, step, None, unroll=False)

    pltpu.sync_copy(out_v, out_hbm.at[:, pl.ds(base, TPW)])


_emb = functools.partial(
    pl.kernel,
    out_type=jax.ShapeDtypeStruct((EMB, NTOK), jnp.float32),
    mesh=plsc.VectorSubcoreMesh(core_axis_name="c", subcore_axis_name="s"),
    scratch_types=[
        pltpu.VMEM((TPW,), jnp.int32),
        pltpu.VMEM((HALF, EMB, 128), jnp.float32),
        pltpu.VMEM_SHARED((NS, HALF, EMB, 128), jnp.float32),
        pltpu.VMEM((EMB, 128), jnp.float32),
        pltpu.VMEM((EMB, TPW), jnp.float32),
        pltpu.SemaphoreType.DMA((HALF,)),
        pltpu.SemaphoreType.DMA((HALF,)),
        pltpu.SemaphoreType.DMA,
    ],
    compiler_params=pltpu.CompilerParams(needs_layout_passes=False),
)(_emb_body)


def kernel(tokens, weight, bias):
    out_t = _emb(tokens, weight.T)
    out = out_t.T
    return (out, out)
